# baseline (speedup 1.0000x reference)
import functools

import jax
import jax.numpy as jnp
from jax.experimental import pallas as pl
from jax.experimental.pallas import tpu as pltpu


def _rvq_body(x_ref, cb_ref, q_ref, codes_ref, *, n_q, n_bins):
    r = x_ref[0]
    qacc = jnp.zeros_like(r)
    k_iota = jax.lax.broadcasted_iota(jnp.int32, (n_bins, r.shape[1]), 0)
    for i in range(n_q):
        cb = cb_ref[i]
        scores = jax.lax.dot_general(
            cb, r, (((1,), (0,)), ((), ())),
            preferred_element_type=jnp.float32,
            precision=jax.lax.Precision.DEFAULT)
        rn = jnp.sum(r * r, axis=0, keepdims=True)
        cn = jnp.sum(cb * cb, axis=1, keepdims=True)
        dist = (rn - 2.0 * scores) + cn
        m = jnp.min(dist, axis=0, keepdims=True)
        idx = jnp.min(jnp.where(dist == m, k_iota, n_bins),
                      axis=0, keepdims=True)
        codes_ref[0, pl.ds(i, 1), :] = idx
        onehot = (k_iota == idx).astype(jnp.float32)
        q = jax.lax.dot_general(
            cb, onehot, (((0,), (0,)), ((), ())),
            preferred_element_type=jnp.float32,
            precision=jax.lax.Precision.HIGHEST)
        qacc = qacc + q
        r = r - q
    q_ref[0] = qacc


def kernel(x, codebooks):
    B, D, T = x.shape
    NQ, K, _ = codebooks.shape
    TB = 256
    grid = (B, T // TB)
    body = functools.partial(_rvq_body, n_q=NQ, n_bins=K)
    q_out, codes_bqt = pl.pallas_call(
        body,
        grid=grid,
        in_specs=[
            pl.BlockSpec((1, D, TB), lambda b, t: (b, 0, t)),
            pl.BlockSpec((NQ, K, D), lambda b, t: (0, 0, 0)),
        ],
        out_specs=[
            pl.BlockSpec((1, D, TB), lambda b, t: (b, 0, t)),
            pl.BlockSpec((1, NQ, TB), lambda b, t: (b, 0, t)),
        ],
        out_shape=[
            jax.ShapeDtypeStruct((B, D, T), jnp.float32),
            jax.ShapeDtypeStruct((B, NQ, T), jnp.int32),
        ],
        compiler_params=pltpu.CompilerParams(
            dimension_semantics=("parallel", "parallel")),
    )(x, codebooks)
    return q_out, jnp.transpose(codes_bqt, (1, 0, 2))


# speedup vs baseline: 1.5378x; 1.5378x over previous
import functools

import jax
import jax.numpy as jnp
from jax.experimental import pallas as pl
from jax.experimental.pallas import tpu as pltpu

_BF16_DOT = dict(preferred_element_type=jnp.float32,
                 precision=jax.lax.Precision.DEFAULT)


def _rvq_body(x_ref, cb_ref, q_ref, codes_ref, hi_s, mid_s, lo_s,
              *, n_q, n_bins):
    @pl.when(jnp.logical_and(pl.program_id(0) == 0, pl.program_id(1) == 0))
    def _init():
        for i in range(n_q):
            cb = cb_ref[i]
            hi = cb.astype(jnp.bfloat16)
            rem = cb - hi.astype(jnp.float32)
            mid = rem.astype(jnp.bfloat16)
            lo = (rem - mid.astype(jnp.float32)).astype(jnp.bfloat16)
            hi_s[i] = hi
            mid_s[i] = mid
            lo_s[i] = lo

    r = x_ref[0]
    qacc = jnp.zeros_like(r)
    k_iota = jax.lax.broadcasted_iota(jnp.int32, (n_bins, r.shape[1]), 0)
    for i in range(n_q):
        cb = cb_ref[i]
        hi = hi_s[i]
        rb = r.astype(jnp.bfloat16)
        scores = jax.lax.dot_general(hi, rb, (((1,), (0,)), ((), ())),
                                     **_BF16_DOT)
        rn = jnp.sum(r * r, axis=0, keepdims=True)
        cn = jnp.sum(cb * cb, axis=1, keepdims=True)
        dist = (rn - 2.0 * scores) + cn
        m = jnp.min(dist, axis=0, keepdims=True)
        idx = jnp.min(jnp.where(dist == m, k_iota, n_bins),
                      axis=0, keepdims=True)
        codes_ref[0, pl.ds(i, 1), :] = idx
        onehot = (k_iota == idx).astype(jnp.bfloat16)
        q = (jax.lax.dot_general(hi, onehot, (((0,), (0,)), ((), ())),
                                 **_BF16_DOT)
             + (jax.lax.dot_general(mid_s[i], onehot, (((0,), (0,)), ((), ())),
                                    **_BF16_DOT)
                + jax.lax.dot_general(lo_s[i], onehot, (((0,), (0,)), ((), ())),
                                      **_BF16_DOT)))
        q = r + (q - r)
        qacc = qacc + q
        r = r - q
    q_ref[0] = qacc


def kernel(x, codebooks):
    B, D, T = x.shape
    NQ, K, _ = codebooks.shape
    TB = 256
    grid = (B, T // TB)
    body = functools.partial(_rvq_body, n_q=NQ, n_bins=K)
    q_out, codes_bqt = pl.pallas_call(
        body,
        grid=grid,
        in_specs=[
            pl.BlockSpec((1, D, TB), lambda b, t: (b, 0, t)),
            pl.BlockSpec((NQ, K, D), lambda b, t: (0, 0, 0)),
        ],
        out_specs=[
            pl.BlockSpec((1, D, TB), lambda b, t: (b, 0, t)),
            pl.BlockSpec((1, NQ, TB), lambda b, t: (b, 0, t)),
        ],
        out_shape=[
            jax.ShapeDtypeStruct((B, D, T), jnp.float32),
            jax.ShapeDtypeStruct((B, NQ, T), jnp.int32),
        ],
        scratch_shapes=[
            pltpu.VMEM((NQ, K, D), jnp.bfloat16),
            pltpu.VMEM((NQ, K, D), jnp.bfloat16),
            pltpu.VMEM((NQ, K, D), jnp.bfloat16),
        ],
        compiler_params=pltpu.CompilerParams(
            dimension_semantics=("arbitrary", "arbitrary")),
    )(x, codebooks)
    return q_out, jnp.transpose(codes_bqt, (1, 0, 2))


# speedup vs baseline: 1.5401x; 1.0015x over previous
import functools

import jax
import jax.numpy as jnp
from jax.experimental import pallas as pl
from jax.experimental.pallas import tpu as pltpu

_BF16_DOT = dict(preferred_element_type=jnp.float32,
                 precision=jax.lax.Precision.DEFAULT)


def _rvq_body(x_ref, cb_ref, q_ref, codes_ref, hi_s, mid_s, lo_s, cn_s,
              *, n_q, n_bins):
    @pl.when(jnp.logical_and(pl.program_id(0) == 0, pl.program_id(1) == 0))
    def _init():
        for i in range(n_q):
            cb = cb_ref[i]
            hi = cb.astype(jnp.bfloat16)
            rem = cb - hi.astype(jnp.float32)
            mid = rem.astype(jnp.bfloat16)
            lo = (rem - mid.astype(jnp.float32)).astype(jnp.bfloat16)
            hi_s[i] = hi
            mid_s[i] = mid
            lo_s[i] = lo
            cn_s[i] = jnp.sum(cb * cb, axis=1, keepdims=True)

    r = x_ref[0]
    qacc = jnp.zeros_like(r)
    k_iota = jax.lax.broadcasted_iota(jnp.int32, (n_bins, r.shape[1]), 0)
    for i in range(n_q):
        hi = hi_s[i]
        rb = r.astype(jnp.bfloat16)
        scores = jax.lax.dot_general(hi, rb, (((1,), (0,)), ((), ())),
                                     **_BF16_DOT)
        rn = jnp.sum(r * r, axis=0, keepdims=True)
        cn = cn_s[i]
        dist = (rn - 2.0 * scores) + cn
        m = jnp.min(dist, axis=0, keepdims=True)
        idx = jnp.min(jnp.where(dist == m, k_iota, n_bins),
                      axis=0, keepdims=True)
        codes_ref[0, pl.ds(i, 1), :] = idx
        onehot = (k_iota == idx).astype(jnp.bfloat16)
        q = (jax.lax.dot_general(hi, onehot, (((0,), (0,)), ((), ())),
                                 **_BF16_DOT)
             + (jax.lax.dot_general(mid_s[i], onehot, (((0,), (0,)), ((), ())),
                                    **_BF16_DOT)
                + jax.lax.dot_general(lo_s[i], onehot, (((0,), (0,)), ((), ())),
                                      **_BF16_DOT)))
        q = r + (q - r)
        qacc = qacc + q
        r = r - q
    q_ref[0] = qacc


def kernel(x, codebooks):
    B, D, T = x.shape
    NQ, K, _ = codebooks.shape
    TB = 256
    grid = (B, T // TB)
    body = functools.partial(_rvq_body, n_q=NQ, n_bins=K)
    q_out, codes_bqt = pl.pallas_call(
        body,
        grid=grid,
        in_specs=[
            pl.BlockSpec((1, D, TB), lambda b, t: (b, 0, t)),
            pl.BlockSpec((NQ, K, D), lambda b, t: (0, 0, 0)),
        ],
        out_specs=[
            pl.BlockSpec((1, D, TB), lambda b, t: (b, 0, t)),
            pl.BlockSpec((1, NQ, TB), lambda b, t: (b, 0, t)),
        ],
        out_shape=[
            jax.ShapeDtypeStruct((B, D, T), jnp.float32),
            jax.ShapeDtypeStruct((B, NQ, T), jnp.int32),
        ],
        scratch_shapes=[
            pltpu.VMEM((NQ, K, D), jnp.bfloat16),
            pltpu.VMEM((NQ, K, D), jnp.bfloat16),
            pltpu.VMEM((NQ, K, D), jnp.bfloat16),
            pltpu.VMEM((NQ, K, 1), jnp.float32),
        ],
        compiler_params=pltpu.CompilerParams(
            dimension_semantics=("arbitrary", "arbitrary")),
    )(x, codebooks)
    return q_out, jnp.transpose(codes_bqt, (1, 0, 2))


# speedup vs baseline: 1.5929x; 1.0343x over previous
import functools

import jax
import jax.numpy as jnp
from jax.experimental import pallas as pl
from jax.experimental.pallas import tpu as pltpu

_BF16_DOT = dict(preferred_element_type=jnp.float32,
                 precision=jax.lax.Precision.DEFAULT)
_NC = 4


def _rvq_body(x_ref, cb_ref, q_ref, codes_ref, hi_s, mid_s, lo_s, cn_s,
              *, n_q, n_bins):
    @pl.when(jnp.logical_and(pl.program_id(0) == 0, pl.program_id(1) == 0))
    def _init():
        for i in range(n_q):
            cb = cb_ref[i]
            hi = cb.astype(jnp.bfloat16)
            rem = cb - hi.astype(jnp.float32)
            mid = rem.astype(jnp.bfloat16)
            lo = (rem - mid.astype(jnp.float32)).astype(jnp.bfloat16)
            hi_s[i] = hi
            mid_s[i] = mid
            lo_s[i] = lo
            cn_s[i] = jnp.sum(cb * cb, axis=1, keepdims=True)

    kc = n_bins // _NC
    r = x_ref[0]
    tb = r.shape[1]
    qacc = jnp.zeros_like(r)
    iotas = [jax.lax.broadcasted_iota(jnp.int32, (kc, tb), 0) + c * kc
             for c in range(_NC)]
    for i in range(n_q):
        rb = r.astype(jnp.bfloat16)
        rn = jnp.sum(r * r, axis=0, keepdims=True)
        ms, ids = [], []
        for c in range(_NC):
            hi_c = hi_s[i, pl.ds(c * kc, kc)]
            s_c = jax.lax.dot_general(hi_c, rb, (((1,), (0,)), ((), ())),
                                      **_BF16_DOT)
            d_c = (rn - 2.0 * s_c) + cn_s[i, pl.ds(c * kc, kc)]
            m_c = jnp.min(d_c, axis=0, keepdims=True)
            i_c = jnp.min(jnp.where(d_c == m_c, iotas[c], n_bins),
                          axis=0, keepdims=True)
            ms.append(m_c)
            ids.append(i_c)
        m = ms[0]
        for c in range(1, _NC):
            m = jnp.minimum(m, ms[c])
        idx = jnp.full_like(ids[0], n_bins)
        for c in range(_NC):
            idx = jnp.minimum(idx, jnp.where(ms[c] == m, ids[c], n_bins))
        codes_ref[0, pl.ds(i, 1), :] = idx
        q = None
        for c in range(_NC):
            oh_c = (iotas[c] == idx).astype(jnp.bfloat16)
            dims = (((0,), (0,)), ((), ()))
            q_c = (jax.lax.dot_general(hi_s[i, pl.ds(c * kc, kc)], oh_c,
                                       dims, **_BF16_DOT)
                   + (jax.lax.dot_general(mid_s[i, pl.ds(c * kc, kc)], oh_c,
                                          dims, **_BF16_DOT)
                      + jax.lax.dot_general(lo_s[i, pl.ds(c * kc, kc)], oh_c,
                                            dims, **_BF16_DOT)))
            q = q_c if q is None else q + q_c
        q = r + (q - r)
        qacc = qacc + q
        r = r - q
    q_ref[0] = qacc


def kernel(x, codebooks):
    B, D, T = x.shape
    NQ, K, _ = codebooks.shape
    TB = 256
    grid = (B, T // TB)
    body = functools.partial(_rvq_body, n_q=NQ, n_bins=K)
    q_out, codes_bqt = pl.pallas_call(
        body,
        grid=grid,
        in_specs=[
            pl.BlockSpec((1, D, TB), lambda b, t: (b, 0, t)),
            pl.BlockSpec((NQ, K, D), lambda b, t: (0, 0, 0)),
        ],
        out_specs=[
            pl.BlockSpec((1, D, TB), lambda b, t: (b, 0, t)),
            pl.BlockSpec((1, NQ, TB), lambda b, t: (b, 0, t)),
        ],
        out_shape=[
            jax.ShapeDtypeStruct((B, D, T), jnp.float32),
            jax.ShapeDtypeStruct((B, NQ, T), jnp.int32),
        ],
        scratch_shapes=[
            pltpu.VMEM((NQ, K, D), jnp.bfloat16),
            pltpu.VMEM((NQ, K, D), jnp.bfloat16),
            pltpu.VMEM((NQ, K, D), jnp.bfloat16),
            pltpu.VMEM((NQ, K, 1), jnp.float32),
        ],
        compiler_params=pltpu.CompilerParams(
            dimension_semantics=("arbitrary", "arbitrary")),
    )(x, codebooks)
    return q_out, jnp.transpose(codes_bqt, (1, 0, 2))


# speedup vs baseline: 2.1762x; 1.3662x over previous
import functools

import jax
import jax.numpy as jnp
from jax.experimental import pallas as pl
from jax.experimental.pallas import tpu as pltpu

_BF16_DOT = dict(preferred_element_type=jnp.float32,
                 precision=jax.lax.Precision.DEFAULT)
_NC = 4


def _rvq_body(x_ref, cb_ref, q_ref, codes_ref, hi_s, mid_s, lo_s, cn_s,
              *, n_q, n_bins):
    @pl.when(jnp.logical_and(pl.program_id(0) == 0, pl.program_id(1) == 0))
    def _init():
        for i in range(n_q):
            cb = cb_ref[i]
            hi = cb.astype(jnp.bfloat16)
            rem = cb - hi.astype(jnp.float32)
            mid = rem.astype(jnp.bfloat16)
            lo = (rem - mid.astype(jnp.float32)).astype(jnp.bfloat16)
            hi_s[i] = hi
            mid_s[i] = mid
            lo_s[i] = lo
            cn_s[i] = jnp.sum(cb * cb, axis=1, keepdims=True)

    kc = n_bins // _NC
    r = x_ref[0]
    tb = r.shape[1]
    qacc = jnp.zeros_like(r)
    iotas = [jax.lax.broadcasted_iota(jnp.int32, (kc, tb), 0) + c * kc
             for c in range(_NC)]
    iota_bf = jax.lax.broadcasted_iota(jnp.int32, (kc, tb), 0).astype(
        jnp.bfloat16)
    for i in range(n_q):
        rb = r.astype(jnp.bfloat16)
        rn = jnp.sum(r * r, axis=0, keepdims=True)
        ms, ids = [], []
        for c in range(_NC):
            hi_c = hi_s[i, pl.ds(c * kc, kc)]
            s_c = jax.lax.dot_general(hi_c, rb, (((1,), (0,)), ((), ())),
                                      **_BF16_DOT)
            d_c = (rn - 2.0 * s_c) + cn_s[i, pl.ds(c * kc, kc)]
            m_c = jnp.min(d_c, axis=0, keepdims=True)
            i_c = jnp.min(jnp.where(d_c == m_c, iotas[c], n_bins),
                          axis=0, keepdims=True)
            ms.append(m_c)
            ids.append(i_c)
        m = ms[0]
        for c in range(1, _NC):
            m = jnp.minimum(m, ms[c])
        idx = jnp.full_like(ids[0], n_bins)
        for c in range(_NC):
            idx = jnp.minimum(idx, jnp.where(ms[c] == m, ids[c], n_bins))
        codes_ref[0, pl.ds(i, 1), :] = idx
        q = None
        for c in range(_NC):
            idxl_bf = (idx - c * kc).astype(jnp.bfloat16)
            oh_c = (iota_bf == idxl_bf).astype(jnp.bfloat16)
            dims = (((0,), (0,)), ((), ()))
            q_c = (jax.lax.dot_general(hi_s[i, pl.ds(c * kc, kc)], oh_c,
                                       dims, **_BF16_DOT)
                   + (jax.lax.dot_general(mid_s[i, pl.ds(c * kc, kc)], oh_c,
                                          dims, **_BF16_DOT)
                      + jax.lax.dot_general(lo_s[i, pl.ds(c * kc, kc)], oh_c,
                                            dims, **_BF16_DOT)))
            q = q_c if q is None else q + q_c
        q = r + (q - r)
        qacc = qacc + q
        r = r - q
    q_ref[0] = qacc


def kernel(x, codebooks):
    B, D, T = x.shape
    NQ, K, _ = codebooks.shape
    TB = 512
    grid = (B, T // TB)
    body = functools.partial(_rvq_body, n_q=NQ, n_bins=K)
    q_out, codes_bqt = pl.pallas_call(
        body,
        grid=grid,
        in_specs=[
            pl.BlockSpec((1, D, TB), lambda b, t: (b, 0, t)),
            pl.BlockSpec((NQ, K, D), lambda b, t: (0, 0, 0)),
        ],
        out_specs=[
            pl.BlockSpec((1, D, TB), lambda b, t: (b, 0, t)),
            pl.BlockSpec((1, NQ, TB), lambda b, t: (b, 0, t)),
        ],
        out_shape=[
            jax.ShapeDtypeStruct((B, D, T), jnp.float32),
            jax.ShapeDtypeStruct((B, NQ, T), jnp.int32),
        ],
        scratch_shapes=[
            pltpu.VMEM((NQ, K, D), jnp.bfloat16),
            pltpu.VMEM((NQ, K, D), jnp.bfloat16),
            pltpu.VMEM((NQ, K, D), jnp.bfloat16),
            pltpu.VMEM((NQ, K, 1), jnp.float32),
        ],
        compiler_params=pltpu.CompilerParams(
            dimension_semantics=("arbitrary", "arbitrary")),
    )(x, codebooks)
    return q_out, jnp.transpose(codes_bqt, (1, 0, 2))


# speedup vs baseline: 2.1820x; 1.0026x over previous
import functools

import jax
import jax.numpy as jnp
from jax.experimental import pallas as pl
from jax.experimental.pallas import tpu as pltpu

_BF16_DOT = dict(preferred_element_type=jnp.float32,
                 precision=jax.lax.Precision.DEFAULT)
_NC = 4


def _rvq_body(x_ref, cb_ref, q_ref, codes_ref, hi2_s, pl3_s, cn_s,
              *, n_q, n_bins):
    kc = n_bins // _NC

    @pl.when(jnp.logical_and(pl.program_id(0) == 0, pl.program_id(1) == 0))
    def _init():
        for i in range(n_q):
            cb = cb_ref[i]
            hi = cb.astype(jnp.bfloat16)
            hif = hi.astype(jnp.float32)
            rem = cb - hif
            mid = rem.astype(jnp.bfloat16)
            lo = (rem - mid.astype(jnp.float32)).astype(jnp.bfloat16)
            hi2_s[i] = (-2.0 * hif).astype(jnp.bfloat16)
            for c in range(_NC):
                s0, s1 = c * kc, (c + 1) * kc
                pl3_s[i, pl.ds((3 * c + 0) * kc, kc)] = hi[s0:s1]
                pl3_s[i, pl.ds((3 * c + 1) * kc, kc)] = mid[s0:s1]
                pl3_s[i, pl.ds((3 * c + 2) * kc, kc)] = lo[s0:s1]
            cn_s[i] = jnp.sum(cb * cb, axis=1, keepdims=True)

    r = x_ref[0]
    tb = r.shape[1]
    qacc = jnp.zeros_like(r)
    iotas = [jax.lax.broadcasted_iota(jnp.int32, (kc, tb), 0) + c * kc
             for c in range(_NC)]
    iota_bf = jax.lax.broadcasted_iota(jnp.int32, (kc, tb), 0).astype(
        jnp.bfloat16)
    for i in range(n_q):
        rb = r.astype(jnp.bfloat16)
        rn = jnp.sum(r * r, axis=0, keepdims=True)
        ms, ids = [], []
        for c in range(_NC):
            h2_c = hi2_s[i, pl.ds(c * kc, kc)]
            s2_c = jax.lax.dot_general(h2_c, rb, (((1,), (0,)), ((), ())),
                                       **_BF16_DOT)
            d_c = (rn + s2_c) + cn_s[i, pl.ds(c * kc, kc)]
            m_c = jnp.min(d_c, axis=0, keepdims=True)
            i_c = jnp.min(jnp.where(d_c == m_c, iotas[c], n_bins),
                          axis=0, keepdims=True)
            ms.append(m_c)
            ids.append(i_c)
        m = ms[0]
        for c in range(1, _NC):
            m = jnp.minimum(m, ms[c])
        idx = jnp.full_like(ids[0], n_bins)
        for c in range(_NC):
            idx = jnp.minimum(idx, jnp.where(ms[c] == m, ids[c], n_bins))
        codes_ref[0, pl.ds(i, 1), :] = idx
        oh_parts = []
        for c in range(_NC):
            idxl_bf = (idx - c * kc).astype(jnp.bfloat16)
            oh_c = (iota_bf == idxl_bf).astype(jnp.bfloat16)
            oh_parts += [oh_c, oh_c, oh_c]
        oh3 = jnp.concatenate(oh_parts, axis=0)
        q = jax.lax.dot_general(pl3_s[i], oh3, (((0,), (0,)), ((), ())),
                                **_BF16_DOT)
        q = r + (q - r)
        qacc = qacc + q
        r = r - q
    q_ref[0] = qacc


def kernel(x, codebooks):
    B, D, T = x.shape
    NQ, K, _ = codebooks.shape
    TB = 512
    grid = (B, T // TB)
    body = functools.partial(_rvq_body, n_q=NQ, n_bins=K)
    q_out, codes_bqt = pl.pallas_call(
        body,
        grid=grid,
        in_specs=[
            pl.BlockSpec((1, D, TB), lambda b, t: (b, 0, t)),
            pl.BlockSpec((NQ, K, D), lambda b, t: (0, 0, 0)),
        ],
        out_specs=[
            pl.BlockSpec((1, D, TB), lambda b, t: (b, 0, t)),
            pl.BlockSpec((1, NQ, TB), lambda b, t: (b, 0, t)),
        ],
        out_shape=[
            jax.ShapeDtypeStruct((B, D, T), jnp.float32),
            jax.ShapeDtypeStruct((B, NQ, T), jnp.int32),
        ],
        scratch_shapes=[
            pltpu.VMEM((NQ, K, D), jnp.bfloat16),
            pltpu.VMEM((NQ, 3 * K, D), jnp.bfloat16),
            pltpu.VMEM((NQ, K, 1), jnp.float32),
        ],
        compiler_params=pltpu.CompilerParams(
            dimension_semantics=("arbitrary", "arbitrary")),
    )(x, codebooks)
    return q_out, jnp.transpose(codes_bqt, (1, 0, 2))


# speedup vs baseline: 2.2882x; 1.0487x over previous
import functools

import jax
import jax.numpy as jnp
from jax.experimental import pallas as pl
from jax.experimental.pallas import tpu as pltpu

_BF16_DOT = dict(preferred_element_type=jnp.float32,
                 precision=jax.lax.Precision.DEFAULT)
_NC = 4


def _rvq_body(x_ref, cb_ref, q_ref, codes_ref, hi2_s, pl3_s, cn_s,
              *, n_q, n_bins):
    kc = n_bins // _NC

    @pl.when(jnp.logical_and(pl.program_id(0) == 0, pl.program_id(1) == 0))
    def _init():
        for i in range(n_q):
            cb = cb_ref[i]
            hi = cb.astype(jnp.bfloat16)
            hif = hi.astype(jnp.float32)
            rem = cb - hif
            mid = rem.astype(jnp.bfloat16)
            lo = (rem - mid.astype(jnp.float32)).astype(jnp.bfloat16)
            hi2_s[i] = (-2.0 * hif).astype(jnp.bfloat16)
            d = cb.shape[1]
            pl3_s[i, :, pl.ds(0, d)] = hi
            pl3_s[i, :, pl.ds(d, d)] = mid
            pl3_s[i, :, pl.ds(2 * d, d)] = lo
            cn_s[i] = jnp.sum(cb * cb, axis=1, keepdims=True)

    r = x_ref[0]
    tb = r.shape[1]
    qacc = jnp.zeros_like(r)
    iotas = [jax.lax.broadcasted_iota(jnp.int32, (kc, tb), 0) + c * kc
             for c in range(_NC)]
    iota_bf = jax.lax.broadcasted_iota(jnp.int32, (kc, tb), 0).astype(
        jnp.bfloat16)
    for i in range(n_q):
        rb = r.astype(jnp.bfloat16)
        rn = jnp.sum(r * r, axis=0, keepdims=True)
        ms, ids = [], []
        for c in range(_NC):
            h2_c = hi2_s[i, pl.ds(c * kc, kc)]
            s2_c = jax.lax.dot_general(h2_c, rb, (((1,), (0,)), ((), ())),
                                       **_BF16_DOT)
            d_c = (rn + s2_c) + cn_s[i, pl.ds(c * kc, kc)]
            m_c = jnp.min(d_c, axis=0, keepdims=True)
            i_c = jnp.min(jnp.where(d_c == m_c, iotas[c], n_bins),
                          axis=0, keepdims=True)
            ms.append(m_c)
            ids.append(i_c)
        m = ms[0]
        for c in range(1, _NC):
            m = jnp.minimum(m, ms[c])
        idx = jnp.full_like(ids[0], n_bins)
        for c in range(_NC):
            idx = jnp.minimum(idx, jnp.where(ms[c] == m, ids[c], n_bins))
        codes_ref[0, pl.ds(i, 1), :] = idx
        oh_parts = []
        for c in range(_NC):
            idxl_bf = (idx - c * kc).astype(jnp.bfloat16)
            oh_parts.append((iota_bf == idxl_bf).astype(jnp.bfloat16))
        oh = jnp.concatenate(oh_parts, axis=0)
        q3 = jax.lax.dot_general(pl3_s[i], oh, (((0,), (0,)), ((), ())),
                                 **_BF16_DOT)
        d_ = r.shape[0]
        q = q3[0:d_] + (q3[d_:2 * d_] + q3[2 * d_:3 * d_])
        q = r + (q - r)
        qacc = qacc + q
        r = r - q
    q_ref[0] = qacc


def kernel(x, codebooks):
    B, D, T = x.shape
    NQ, K, _ = codebooks.shape
    TB = 512
    grid = (B, T // TB)
    body = functools.partial(_rvq_body, n_q=NQ, n_bins=K)
    q_out, codes_bqt = pl.pallas_call(
        body,
        grid=grid,
        in_specs=[
            pl.BlockSpec((1, D, TB), lambda b, t: (b, 0, t)),
            pl.BlockSpec((NQ, K, D), lambda b, t: (0, 0, 0)),
        ],
        out_specs=[
            pl.BlockSpec((1, D, TB), lambda b, t: (b, 0, t)),
            pl.BlockSpec((1, NQ, TB), lambda b, t: (b, 0, t)),
        ],
        out_shape=[
            jax.ShapeDtypeStruct((B, D, T), jnp.float32),
            jax.ShapeDtypeStruct((B, NQ, T), jnp.int32),
        ],
        scratch_shapes=[
            pltpu.VMEM((NQ, K, D), jnp.bfloat16),
            pltpu.VMEM((NQ, K, 3 * D), jnp.bfloat16),
            pltpu.VMEM((NQ, K, 1), jnp.float32),
        ],
        compiler_params=pltpu.CompilerParams(
            dimension_semantics=("arbitrary", "arbitrary")),
    )(x, codebooks)
    return q_out, jnp.transpose(codes_bqt, (1, 0, 2))


# speedup vs baseline: 2.5478x; 1.1135x over previous
import functools

import jax
import jax.numpy as jnp
from jax.experimental import pallas as pl
from jax.experimental.pallas import tpu as pltpu

_BF16_DOT = dict(preferred_element_type=jnp.float32,
                 precision=jax.lax.Precision.DEFAULT)
_NC = 4


def _rvq_body(x_ref, cb_ref, q_ref, codes_ref, hi2_s, pl3_s, cn_s,
              *, n_q, n_bins):
    kc = n_bins // _NC

    @pl.when(jnp.logical_and(pl.program_id(0) == 0, pl.program_id(1) == 0))
    def _init():
        for i in range(n_q):
            cb = cb_ref[i]
            hi = cb.astype(jnp.bfloat16)
            hif = hi.astype(jnp.float32)
            rem = cb - hif
            mid = rem.astype(jnp.bfloat16)
            lo = (rem - mid.astype(jnp.float32)).astype(jnp.bfloat16)
            hi2_s[i] = (-2.0 * hif).astype(jnp.bfloat16)
            d = cb.shape[1]
            pl3_s[i, :, pl.ds(0, d)] = hi
            pl3_s[i, :, pl.ds(d, d)] = mid
            pl3_s[i, :, pl.ds(2 * d, d)] = lo
            cn_s[i] = jnp.sum(cb * cb, axis=1, keepdims=True)

    r = x_ref[0]
    tb = r.shape[1]
    qacc = jnp.zeros_like(r)
    iotas = [jax.lax.broadcasted_iota(jnp.int32, (kc, tb), 0) + c * kc
             for c in range(_NC)]
    iota_bf = jax.lax.broadcasted_iota(jnp.int32, (kc, tb), 0).astype(
        jnp.bfloat16)
    for i in range(n_q):
        rb = r.astype(jnp.bfloat16)
        rn = jnp.sum(r * r, axis=0, keepdims=True)
        ms, ids = [], []
        for c in range(_NC):
            h2_c = hi2_s[i, pl.ds(c * kc, kc)]
            s2_c = jax.lax.dot_general(h2_c, rb, (((1,), (0,)), ((), ())),
                                       **_BF16_DOT)
            d_c = (rn + s2_c) + cn_s[i, pl.ds(c * kc, kc)]
            m_c = jnp.min(d_c, axis=0, keepdims=True)
            i_c = jnp.min(jnp.where(d_c == m_c, iotas[c], n_bins),
                          axis=0, keepdims=True)
            ms.append(m_c)
            ids.append(i_c)
        m = ms[0]
        for c in range(1, _NC):
            m = jnp.minimum(m, ms[c])
        idx = jnp.full_like(ids[0], n_bins)
        for c in range(_NC):
            idx = jnp.minimum(idx, jnp.where(ms[c] == m, ids[c], n_bins))
        codes_ref[0, pl.ds(i, 1), :] = idx
        oh_parts = []
        for c in range(_NC):
            idxl_bf = (idx - c * kc).astype(jnp.bfloat16)
            oh_parts.append((iota_bf == idxl_bf).astype(jnp.bfloat16))
        oh = jnp.concatenate(oh_parts, axis=0)
        q3 = jax.lax.dot_general(pl3_s[i], oh, (((0,), (0,)), ((), ())),
                                 **_BF16_DOT)
        d_ = r.shape[0]
        q = q3[0:d_] + (q3[d_:2 * d_] + q3[2 * d_:3 * d_])
        q = r + (q - r)
        qacc = qacc + q
        r = r - q
    q_ref[0] = qacc


def kernel(x, codebooks):
    B, D, T = x.shape
    NQ, K, _ = codebooks.shape
    TB = 1024
    grid = (B, T // TB)
    body = functools.partial(_rvq_body, n_q=NQ, n_bins=K)
    q_out, codes_bqt = pl.pallas_call(
        body,
        grid=grid,
        in_specs=[
            pl.BlockSpec((1, D, TB), lambda b, t: (b, 0, t)),
            pl.BlockSpec((NQ, K, D), lambda b, t: (0, 0, 0)),
        ],
        out_specs=[
            pl.BlockSpec((1, D, TB), lambda b, t: (b, 0, t)),
            pl.BlockSpec((1, NQ, TB), lambda b, t: (b, 0, t)),
        ],
        out_shape=[
            jax.ShapeDtypeStruct((B, D, T), jnp.float32),
            jax.ShapeDtypeStruct((B, NQ, T), jnp.int32),
        ],
        scratch_shapes=[
            pltpu.VMEM((NQ, K, D), jnp.bfloat16),
            pltpu.VMEM((NQ, K, 3 * D), jnp.bfloat16),
            pltpu.VMEM((NQ, K, 1), jnp.float32),
        ],
        compiler_params=pltpu.CompilerParams(
            dimension_semantics=("arbitrary", "arbitrary")),
    )(x, codebooks)
    return q_out, jnp.transpose(codes_bqt, (1, 0, 2))


# speedup vs baseline: 2.5662x; 1.0072x over previous
import functools

import jax
import jax.numpy as jnp
from jax.experimental import pallas as pl
from jax.experimental.pallas import tpu as pltpu

_BF16_DOT = dict(preferred_element_type=jnp.float32,
                 precision=jax.lax.Precision.DEFAULT)
_NC = 4


def _rvq_body(x_ref, cb_ref, q_ref, codes_ref, hi2_s, pl3_s, cn_s,
              *, n_q, n_bins):
    kc = n_bins // _NC

    @pl.when(jnp.logical_and(pl.program_id(0) == 0, pl.program_id(1) == 0))
    def _init():
        for i in range(n_q):
            cb = cb_ref[i]
            hi = cb.astype(jnp.bfloat16)
            hif = hi.astype(jnp.float32)
            rem = cb - hif
            mid = rem.astype(jnp.bfloat16)
            lo = (rem - mid.astype(jnp.float32)).astype(jnp.bfloat16)
            hi2_s[i] = (-2.0 * hif).astype(jnp.bfloat16)
            d = cb.shape[1]
            pl3_s[i, :, pl.ds(0, d)] = hi
            pl3_s[i, :, pl.ds(d, d)] = mid
            pl3_s[i, :, pl.ds(2 * d, d)] = lo
            cn_s[i] = jnp.sum(cb * cb, axis=1, keepdims=True)

    x_blk = x_ref[0]
    d_ = x_blk.shape[0]
    tb = x_blk.shape[1]
    th = tb // 2
    iotas = [jax.lax.broadcasted_iota(jnp.int32, (kc, th), 0) + c * kc
             for c in range(_NC)]
    iota_bf = jax.lax.broadcasted_iota(jnp.int32, (kc, th), 0).astype(
        jnp.bfloat16)
    rs = [x_blk[:, 0:th], x_blk[:, th:tb]]
    qaccs = [jnp.zeros((d_, th), jnp.float32)] * 2
    for i in range(n_q):
        rbs = [r.astype(jnp.bfloat16) for r in rs]
        rns = [jnp.sum(r * r, axis=0, keepdims=True) for r in rs]
        ms = [[], []]
        ids = [[], []]
        for c in range(_NC):
            h2_c = hi2_s[i, pl.ds(c * kc, kc)]
            cn_c = cn_s[i, pl.ds(c * kc, kc)]
            for h in range(2):
                s2_c = jax.lax.dot_general(h2_c, rbs[h],
                                           (((1,), (0,)), ((), ())),
                                           **_BF16_DOT)
                d_c = (rns[h] + s2_c) + cn_c
                m_c = jnp.min(d_c, axis=0, keepdims=True)
                i_c = jnp.min(jnp.where(d_c == m_c, iotas[c], n_bins),
                              axis=0, keepdims=True)
                ms[h].append(m_c)
                ids[h].append(i_c)
        idxs = []
        for h in range(2):
            m = ms[h][0]
            for c in range(1, _NC):
                m = jnp.minimum(m, ms[h][c])
            idx = jnp.full_like(ids[h][0], n_bins)
            for c in range(_NC):
                idx = jnp.minimum(idx, jnp.where(ms[h][c] == m, ids[h][c],
                                                 n_bins))
            codes_ref[0, pl.ds(i, 1), pl.ds(h * th, th)] = idx
            idxs.append(idx)
        for h in range(2):
            oh_parts = []
            for c in range(_NC):
                idxl_bf = (idxs[h] - c * kc).astype(jnp.bfloat16)
                oh_parts.append((iota_bf == idxl_bf).astype(jnp.bfloat16))
            oh = jnp.concatenate(oh_parts, axis=0)
            q3 = jax.lax.dot_general(pl3_s[i], oh, (((0,), (0,)), ((), ())),
                                     **_BF16_DOT)
            q = q3[0:d_] + (q3[d_:2 * d_] + q3[2 * d_:3 * d_])
            r = rs[h]
            q = r + (q - r)
            qaccs[h] = qaccs[h] + q
            rs[h] = r - q
    q_ref[0, :, pl.ds(0, th)] = qaccs[0]
    q_ref[0, :, pl.ds(th, th)] = qaccs[1]


def kernel(x, codebooks):
    B, D, T = x.shape
    NQ, K, _ = codebooks.shape
    TB = 1024
    grid = (B, T // TB)
    body = functools.partial(_rvq_body, n_q=NQ, n_bins=K)
    q_out, codes_bqt = pl.pallas_call(
        body,
        grid=grid,
        in_specs=[
            pl.BlockSpec((1, D, TB), lambda b, t: (b, 0, t)),
            pl.BlockSpec((NQ, K, D), lambda b, t: (0, 0, 0)),
        ],
        out_specs=[
            pl.BlockSpec((1, D, TB), lambda b, t: (b, 0, t)),
            pl.BlockSpec((1, NQ, TB), lambda b, t: (b, 0, t)),
        ],
        out_shape=[
            jax.ShapeDtypeStruct((B, D, T), jnp.float32),
            jax.ShapeDtypeStruct((B, NQ, T), jnp.int32),
        ],
        scratch_shapes=[
            pltpu.VMEM((NQ, K, D), jnp.bfloat16),
            pltpu.VMEM((NQ, K, 3 * D), jnp.bfloat16),
            pltpu.VMEM((NQ, K, 1), jnp.float32),
        ],
        compiler_params=pltpu.CompilerParams(
            dimension_semantics=("arbitrary", "arbitrary")),
    )(x, codebooks)
    return q_out, jnp.transpose(codes_bqt, (1, 0, 2))
